# Initial kernel scaffold; baseline (speedup 1.0000x reference)
import jax
import jax.numpy as jnp
from jax.experimental import pallas as pl


def kernel(embeddings, target):
    raise NotImplementedError("write your pallas kernel here")


# speedup vs baseline: 1971.6511x; 1971.6511x over previous
import jax
import jax.numpy as jnp
from jax.experimental import pallas as pl

_B = 2048
_D = 128
_MARGIN = 1.0
_BLK = 256
_NSTEPS = _B // _BLK


def _triplet_kernel(eb_ref, et_ref, labc_ref, labr_ref, sum_ref, cnt_ref):
    i = pl.program_id(0)

    @pl.when(i == 0)
    def _():
        sum_ref[...] = jnp.zeros_like(sum_ref)
        cnt_ref[...] = jnp.zeros_like(cnt_ref)

    eb = eb_ref[...]
    et = et_ref[...]
    sq = jnp.sum(et * et, axis=0, keepdims=True)
    g = jnp.dot(eb, et, preferred_element_type=jnp.float32)
    h = sq - 2.0 * g

    labc = labc_ref[...]
    labr = labr_ref[...]
    diffm = labc != labr

    inf = jnp.float32(jnp.inf)
    neg = jnp.min(jnp.where(diffm, h, inf), axis=1, keepdims=True)
    has_neg = jnp.any(diffm, axis=1, keepdims=True)
    neg = jnp.where(has_neg, neg, h[:, 0:1])

    rows = i * _BLK + jax.lax.broadcasted_iota(jnp.int32, (_BLK, _B), 0)
    cols = jax.lax.broadcasted_iota(jnp.int32, (_BLK, _B), 1)
    pos = jnp.logical_and(jnp.logical_not(diffm), cols > rows)

    losses = jnp.maximum(h - neg + _MARGIN, 0.0)
    sum_ref[...] += jnp.sum(jnp.where(pos, losses, 0.0), keepdims=True)
    cnt_ref[...] += jnp.sum(pos.astype(jnp.int32), keepdims=True)

    @pl.when(i == _NSTEPS - 1)
    def _():
        sum_ref[...] = sum_ref[...] / cnt_ref[...].astype(jnp.float32)


def kernel(embeddings, target):
    et = embeddings.T
    labc = target.reshape(_B, 1)
    labr = target.reshape(1, _B)
    out_sum, out_cnt = pl.pallas_call(
        _triplet_kernel,
        grid=(_NSTEPS,),
        in_specs=[
            pl.BlockSpec((_BLK, _D), lambda i: (i, 0)),
            pl.BlockSpec((_D, _B), lambda i: (0, 0)),
            pl.BlockSpec((_BLK, 1), lambda i: (i, 0)),
            pl.BlockSpec((1, _B), lambda i: (0, 0)),
        ],
        out_specs=[
            pl.BlockSpec((1, 1), lambda i: (0, 0)),
            pl.BlockSpec((1, 1), lambda i: (0, 0)),
        ],
        out_shape=[
            jax.ShapeDtypeStruct((1, 1), jnp.float32),
            jax.ShapeDtypeStruct((1, 1), jnp.int32),
        ],
    )(embeddings, et, labc, labr)
    return (out_sum[0, 0], out_cnt[0, 0])


# speedup vs baseline: 2057.5520x; 1.0436x over previous
import jax
import jax.numpy as jnp
from jax.experimental import pallas as pl
from jax.experimental.pallas import tpu as pltpu

_B = 2048
_D = 128
_MARGIN = 1.0
_BLK = 256
_NSTEPS = _B // _BLK


def _triplet_kernel(eb_ref, et_ref, labc_ref, labr_ref, sum_ref, cnt_ref,
                    sq_ref):
    i = pl.program_id(0)
    et = et_ref[...]

    @pl.when(i == 0)
    def _():
        sum_ref[...] = jnp.zeros_like(sum_ref)
        cnt_ref[...] = jnp.zeros_like(cnt_ref)
        sq_ref[...] = jnp.sum(et * et, axis=0, keepdims=True)

    ebm2 = eb_ref[...] * jnp.float32(-2.0)
    h = sq_ref[...] + jnp.dot(ebm2, et,
                              preferred_element_type=jnp.float32)

    labc = labc_ref[...]
    labr = labr_ref[...]
    diffm = labc != labr

    inf = jnp.float32(jnp.inf)
    neg = jnp.min(jnp.where(diffm, h, inf), axis=1, keepdims=True)
    neg = jnp.where(neg < inf, neg, h[:, 0:1])
    negp = neg - _MARGIN

    colv = jax.lax.broadcasted_iota(jnp.int32, (1, _B), 1)
    rowv = i * _BLK + jax.lax.broadcasted_iota(jnp.int32, (_BLK, 1), 0)
    pos = jnp.logical_and(jnp.logical_not(diffm), colv > rowv)

    losses = jnp.maximum(h - negp, 0.0)
    sum_ref[...] += jnp.sum(jnp.where(pos, losses, 0.0), keepdims=True)
    cnt_ref[...] += jnp.sum(pos.astype(jnp.int32), keepdims=True)

    @pl.when(i == _NSTEPS - 1)
    def _():
        sum_ref[...] = sum_ref[...] / cnt_ref[...].astype(jnp.float32)


def kernel(embeddings, target):
    et = embeddings.T
    labc = target.reshape(_B, 1)
    labr = target.reshape(1, _B)
    out_sum, out_cnt = pl.pallas_call(
        _triplet_kernel,
        grid=(_NSTEPS,),
        in_specs=[
            pl.BlockSpec((_BLK, _D), lambda i: (i, 0)),
            pl.BlockSpec((_D, _B), lambda i: (0, 0)),
            pl.BlockSpec((_BLK, 1), lambda i: (i, 0)),
            pl.BlockSpec((1, _B), lambda i: (0, 0)),
        ],
        out_specs=[
            pl.BlockSpec((1, 1), lambda i: (0, 0)),
            pl.BlockSpec((1, 1), lambda i: (0, 0)),
        ],
        out_shape=[
            jax.ShapeDtypeStruct((1, 1), jnp.float32),
            jax.ShapeDtypeStruct((1, 1), jnp.int32),
        ],
        scratch_shapes=[pltpu.VMEM((1, _B), jnp.float32)],
    )(embeddings, et, labc, labr)
    return (out_sum[0, 0], out_cnt[0, 0])


# speedup vs baseline: 2285.2393x; 1.1107x over previous
import jax
import jax.numpy as jnp
from jax.experimental import pallas as pl
from jax.experimental.pallas import tpu as pltpu

_B = 2048
_D = 128
_MARGIN = 1.0
_BLK = 512
_NSTEPS = _B // _BLK


def _triplet_kernel(eb_ref, et_ref, labc_ref, labr_ref, sum_ref, cnt_ref,
                    sq_ref):
    i = pl.program_id(0)
    et = et_ref[...]

    @pl.when(i == 0)
    def _():
        sum_ref[...] = jnp.zeros_like(sum_ref)
        cnt_ref[...] = jnp.zeros_like(cnt_ref)
        sq_ref[...] = jnp.sum(et * et, axis=0, keepdims=True)

    ebm2 = eb_ref[...] * jnp.float32(-2.0)
    h = sq_ref[...] + jnp.dot(ebm2, et,
                              preferred_element_type=jnp.float32)

    labc = labc_ref[...]
    labr = labr_ref[...]
    diffm = labc != labr

    inf = jnp.float32(jnp.inf)
    neg = jnp.min(jnp.where(diffm, h, inf), axis=1, keepdims=True)
    neg = jnp.where(neg < inf, neg, h[:, 0:1])
    negp = neg - _MARGIN

    colv = jax.lax.broadcasted_iota(jnp.int32, (1, _B), 1)
    rowv = i * _BLK + jax.lax.broadcasted_iota(jnp.int32, (_BLK, 1), 0)
    pos = jnp.logical_and(jnp.logical_not(diffm), colv > rowv)

    losses = jnp.maximum(h - negp, 0.0)
    sum_ref[...] += jnp.sum(jnp.where(pos, losses, 0.0), keepdims=True)
    cnt_ref[...] += jnp.sum(pos.astype(jnp.int32), keepdims=True)

    @pl.when(i == _NSTEPS - 1)
    def _():
        sum_ref[...] = sum_ref[...] / cnt_ref[...].astype(jnp.float32)


def kernel(embeddings, target):
    et = embeddings.T
    labc = target.reshape(_B, 1)
    labr = target.reshape(1, _B)
    out_sum, out_cnt = pl.pallas_call(
        _triplet_kernel,
        grid=(_NSTEPS,),
        in_specs=[
            pl.BlockSpec((_BLK, _D), lambda i: (i, 0)),
            pl.BlockSpec((_D, _B), lambda i: (0, 0)),
            pl.BlockSpec((_BLK, 1), lambda i: (i, 0)),
            pl.BlockSpec((1, _B), lambda i: (0, 0)),
        ],
        out_specs=[
            pl.BlockSpec((1, 1), lambda i: (0, 0)),
            pl.BlockSpec((1, 1), lambda i: (0, 0)),
        ],
        out_shape=[
            jax.ShapeDtypeStruct((1, 1), jnp.float32),
            jax.ShapeDtypeStruct((1, 1), jnp.int32),
        ],
        scratch_shapes=[pltpu.VMEM((1, _B), jnp.float32)],
    )(embeddings, et, labc, labr)
    return (out_sum[0, 0], out_cnt[0, 0])


# speedup vs baseline: 2420.2191x; 1.0591x over previous
import jax
import jax.numpy as jnp
from jax.experimental import pallas as pl
from jax.experimental.pallas import tpu as pltpu

_B = 2048
_D = 128
_MARGIN = 1.0
_BLK = 1024
_NSTEPS = _B // _BLK


def _triplet_kernel(eb_ref, et_ref, labc_ref, labr_ref, sum_ref, cnt_ref,
                    sq_ref):
    i = pl.program_id(0)
    et = et_ref[...]

    @pl.when(i == 0)
    def _():
        sum_ref[...] = jnp.zeros_like(sum_ref)
        cnt_ref[...] = jnp.zeros_like(cnt_ref)
        sq_ref[...] = jnp.sum(et * et, axis=0, keepdims=True)

    ebm2 = eb_ref[...] * jnp.float32(-2.0)
    h = sq_ref[...] + jnp.dot(ebm2, et,
                              preferred_element_type=jnp.float32)

    labc = labc_ref[...]
    labr = labr_ref[...]
    diffm = labc != labr

    inf = jnp.float32(jnp.inf)
    neg = jnp.min(jnp.where(diffm, h, inf), axis=1, keepdims=True)
    neg = jnp.where(neg < inf, neg, h[:, 0:1])
    negp = neg - _MARGIN

    colv = jax.lax.broadcasted_iota(jnp.int32, (1, _B), 1)
    rowv = i * _BLK + jax.lax.broadcasted_iota(jnp.int32, (_BLK, 1), 0)
    pos = jnp.logical_and(jnp.logical_not(diffm), colv > rowv)

    losses = jnp.maximum(h - negp, 0.0)
    sum_ref[...] += jnp.sum(jnp.where(pos, losses, 0.0), keepdims=True)
    cnt_ref[...] += jnp.sum(pos.astype(jnp.int32), keepdims=True)

    @pl.when(i == _NSTEPS - 1)
    def _():
        sum_ref[...] = sum_ref[...] / cnt_ref[...].astype(jnp.float32)


def kernel(embeddings, target):
    et = embeddings.T
    labc = target.reshape(_B, 1)
    labr = target.reshape(1, _B)
    out_sum, out_cnt = pl.pallas_call(
        _triplet_kernel,
        grid=(_NSTEPS,),
        in_specs=[
            pl.BlockSpec((_BLK, _D), lambda i: (i, 0)),
            pl.BlockSpec((_D, _B), lambda i: (0, 0)),
            pl.BlockSpec((_BLK, 1), lambda i: (i, 0)),
            pl.BlockSpec((1, _B), lambda i: (0, 0)),
        ],
        out_specs=[
            pl.BlockSpec((1, 1), lambda i: (0, 0)),
            pl.BlockSpec((1, 1), lambda i: (0, 0)),
        ],
        out_shape=[
            jax.ShapeDtypeStruct((1, 1), jnp.float32),
            jax.ShapeDtypeStruct((1, 1), jnp.int32),
        ],
        scratch_shapes=[pltpu.VMEM((1, _B), jnp.float32)],
    )(embeddings, et, labc, labr)
    return (out_sum[0, 0], out_cnt[0, 0])
